# Initial kernel scaffold; baseline (speedup 1.0000x reference)
import jax
import jax.numpy as jnp
from jax.experimental import pallas as pl


def kernel(feats, edge_index, graph_ids, emb_table, W_mlp, b_mlp, W_reg):
    raise NotImplementedError("write your pallas kernel here")


# speedup vs baseline: 14.0705x; 14.0705x over previous
import functools

import jax
import jax.numpy as jnp
from jax import lax
from jax.experimental import pallas as pl
from jax.experimental.pallas import tpu as pltpu
from jax.experimental.pallas import tpu_sc as plsc

N = 10000
E = 320000
H = 128
V = 128
G = 64
OUT = 1

NC = 2
NS = 16
NW = NC * NS
CHUNK = 128
NCH = 80
EPP = NCH * CHUNK
E_PAD = NW * EPP
CROWS = 10016
CSZ = CROWS * V
STRIPE = CSZ // NS
ZB = STRIPE // 4
GRP = 8


def _sc_hist_body(src_hbm, dst_hbm, feats_hbm, zeros_hbm, out_hbm,
                  src_v, dst_v, fv_v, ones_v, cs, sem):
    c = lax.axis_index("c")
    s = lax.axis_index("s")
    wid = c * NS + s

    pltpu.sync_copy(src_hbm.at[pl.ds(wid * NCH, NCH)], src_v)
    pltpu.sync_copy(dst_hbm.at[pl.ds(wid * NCH, NCH)], dst_v)

    off = s * STRIPE
    pltpu.sync_copy(zeros_hbm.at[pl.ds(off, STRIPE)], cs.at[pl.ds(off, STRIPE)])

    for u in range(CHUNK // 16):
        ones_v[pl.ds(u * 16, 16)] = jnp.full((16,), 1.0, jnp.float32)

    def gather_grp(g, carry):
        cps = []
        for u in range(GRP):
            j = g * GRP + u
            cps.append(pltpu.async_copy(
                feats_hbm.at[src_v.at[j]], fv_v.at[j], sem))
        for cp in cps:
            cp.wait()
        return carry
    lax.fori_loop(0, NCH // GRP, gather_grp, 0)

    def idx_row(j, carry):
        for k in range(CHUNK // 16):
            sl = pl.ds(k * 16, 16)
            dst_v[j, sl] = dst_v[j, sl] * V + fv_v[j, sl]
        return carry
    lax.fori_loop(0, NCH, idx_row, 0)

    plsc.subcore_barrier()

    def scat_grp(g, carry):
        cps = []
        for u in range(GRP):
            j = g * GRP + u
            cps.append(pltpu.async_copy(
                ones_v, cs.at[dst_v.at[j]], sem, add=True))
        for cp in cps:
            cp.wait()
        return carry
    lax.fori_loop(0, NCH // GRP, scat_grp, 0)

    plsc.subcore_barrier()

    pltpu.sync_copy(cs.at[pl.ds(off, STRIPE)], out_hbm.at[c, pl.ds(off, STRIPE)])


@functools.cache
def _get_sc_hist():
    return pl.kernel(
        _sc_hist_body,
        out_type=jax.ShapeDtypeStruct((NC, CSZ), jnp.float32),
        mesh=plsc.VectorSubcoreMesh(core_axis_name="c", subcore_axis_name="s"),
        scratch_types=[
            pltpu.VMEM((NCH, CHUNK), jnp.int32),
            pltpu.VMEM((NCH, CHUNK), jnp.int32),
            pltpu.VMEM((NCH, CHUNK), jnp.int32),
            pltpu.VMEM((CHUNK,), jnp.float32),
            pltpu.VMEM_SHARED((CSZ,), jnp.float32),
            pltpu.SemaphoreType.DMA,
        ],
    )


def _tc_body(c_ref, f_ref, g_ref, emb_ref, wm_ref, bm_ref, wr_ref,
             out_ref, pooled):
    i = pl.program_id(0)
    bn = f_ref.shape[2]
    feats_b = f_ref[0, 0, :]
    oh = (feats_b[:, None]
          == lax.broadcasted_iota(jnp.int32, (bn, V), 1)).astype(jnp.float32)
    cb = c_ref[0] + c_ref[1] + oh
    a = jnp.dot(emb_ref[...], wm_ref[...], preferred_element_type=jnp.float32, precision=lax.Precision.HIGHEST)
    y = jnp.maximum(
        jnp.dot(cb, a, preferred_element_type=jnp.float32, precision=lax.Precision.HIGHEST) + bm_ref[...], 0.0)
    gid_b = g_ref[0, 0, :]
    gm = (gid_b[None, :]
          == lax.broadcasted_iota(jnp.int32, (G, bn), 0)).astype(jnp.float32)
    p = jnp.dot(gm, y, preferred_element_type=jnp.float32, precision=lax.Precision.HIGHEST)

    @pl.when(i == 0)
    def _():
        pooled[...] = p

    @pl.when(i > 0)
    def _():
        pooled[...] += p

    @pl.when(i == pl.num_programs(0) - 1)
    def _():
        out_ref[...] = jnp.dot(pooled[...], wr_ref[...],
                               preferred_element_type=jnp.float32, precision=lax.Precision.HIGHEST)


def kernel(feats, edge_index, graph_ids, emb_table, W_mlp, b_mlp, W_reg):
    pad = E_PAD - E
    src = jnp.concatenate(
        [edge_index[0], jnp.zeros((pad,), jnp.int32)]).reshape(NW * NCH, CHUNK)
    dst = jnp.concatenate(
        [edge_index[1], jnp.full((pad,), N, jnp.int32)]).reshape(NW * NCH, CHUNK)

    craw = _get_sc_hist()(src, dst, feats, jnp.zeros((CSZ,), jnp.float32))
    c2 = craw.reshape(NC, CROWS, V)

    rpad = CROWS - N
    feats_p = jnp.concatenate(
        [feats, jnp.zeros((rpad,), jnp.int32)]).reshape(2, 1, CROWS // 2)
    gids_p = jnp.concatenate(
        [graph_ids, jnp.full((rpad,), G, jnp.int32)]).reshape(2, 1, CROWS // 2)

    nb = 2
    bn = CROWS // nb
    out = pl.pallas_call(
        _tc_body,
        grid=(nb,),
        in_specs=[
            pl.BlockSpec((NC, bn, V), lambda i: (0, i, 0)),
            pl.BlockSpec((1, 1, bn), lambda i: (i, 0, 0)),
            pl.BlockSpec((1, 1, bn), lambda i: (i, 0, 0)),
            pl.BlockSpec((V, H), lambda i: (0, 0)),
            pl.BlockSpec((H, H), lambda i: (0, 0)),
            pl.BlockSpec((1, H), lambda i: (0, 0)),
            pl.BlockSpec((H, OUT), lambda i: (0, 0)),
        ],
        out_specs=pl.BlockSpec((G, OUT), lambda i: (0, 0)),
        out_shape=jax.ShapeDtypeStruct((G, OUT), jnp.float32),
        scratch_shapes=[pltpu.VMEM((G, H), jnp.float32)],
    )(c2, feats_p, gids_p, emb_table, W_mlp, b_mlp.reshape(1, H), W_reg)
    return out


# speedup vs baseline: 22.1391x; 1.5734x over previous
import functools

import jax
import jax.numpy as jnp
from jax import lax
from jax.experimental import pallas as pl
from jax.experimental.pallas import tpu as pltpu
from jax.experimental.pallas import tpu_sc as plsc

N = 10000
E = 320000
H = 128
V = 128
G = 64
OUT = 1

NC = 2
NS = 16
NW = NC * NS
CHUNK = 128
NCH = 80
EPP = NCH * CHUNK
E_PAD = NW * EPP
CROWS = 10016
CSZ = CROWS * V
STRIPE = CSZ // NS
ZB = STRIPE // 4
GRP = 8


def _sc_hist_body(src_hbm, dst_hbm, feats_hbm, zeros_hbm, out_hbm,
                  src_v, dst_v, feats_v, idx_v, ones_v, cs, sem):
    c = lax.axis_index("c")
    s = lax.axis_index("s")
    wid = c * NS + s

    pltpu.sync_copy(src_hbm.at[pl.ds(wid * EPP, EPP)], src_v)
    pltpu.sync_copy(dst_hbm.at[pl.ds(wid * EPP, EPP)], dst_v)
    pltpu.sync_copy(feats_hbm, feats_v)

    off = s * STRIPE
    pltpu.sync_copy(zeros_hbm.at[pl.ds(off, STRIPE)], cs.at[pl.ds(off, STRIPE)])

    for u in range(CHUNK // 16):
        ones_v[pl.ds(u * 16, 16)] = jnp.full((16,), 1.0, jnp.float32)

    def idx_row(j, carry):
        for k in range(CHUNK // 16):
            base = pl.ds(j * CHUNK + k * 16, 16)
            fv = plsc.load_gather(feats_v, [src_v[base]])
            idx_v[j, pl.ds(k * 16, 16)] = dst_v[base] * V + fv
        return carry
    lax.fori_loop(0, NCH, idx_row, 0)

    plsc.subcore_barrier()

    def scat_grp(g, carry):
        cps = []
        for u in range(GRP):
            j = g * GRP + u
            cps.append(pltpu.async_copy(
                ones_v, cs.at[idx_v.at[j]], sem, add=True))
        for cp in cps:
            cp.wait()
        return carry
    lax.fori_loop(0, NCH // GRP, scat_grp, 0)

    plsc.subcore_barrier()

    pltpu.sync_copy(cs.at[pl.ds(off, STRIPE)], out_hbm.at[c, pl.ds(off, STRIPE)])


@functools.cache
def _get_sc_hist():
    return pl.kernel(
        _sc_hist_body,
        out_type=jax.ShapeDtypeStruct((NC, CSZ), jnp.float32),
        mesh=plsc.VectorSubcoreMesh(core_axis_name="c", subcore_axis_name="s"),
        compiler_params=pltpu.CompilerParams(needs_layout_passes=False),
        scratch_types=[
            pltpu.VMEM((EPP,), jnp.int32),
            pltpu.VMEM((EPP,), jnp.int32),
            pltpu.VMEM((N,), jnp.int32),
            pltpu.VMEM((NCH, CHUNK), jnp.int32),
            pltpu.VMEM((CHUNK,), jnp.float32),
            pltpu.VMEM_SHARED((CSZ,), jnp.float32),
            pltpu.SemaphoreType.DMA,
        ],
    )


def _tc_body(c_ref, f_ref, g_ref, emb_ref, wm_ref, bm_ref, wr_ref,
             out_ref, pooled):
    i = pl.program_id(0)
    bn = f_ref.shape[2]
    feats_b = f_ref[0, 0, :]
    oh = (feats_b[:, None]
          == lax.broadcasted_iota(jnp.int32, (bn, V), 1)).astype(jnp.float32)
    cb = c_ref[0] + c_ref[1] + oh
    a = jnp.dot(emb_ref[...], wm_ref[...], preferred_element_type=jnp.float32, precision=lax.Precision.HIGHEST)
    y = jnp.maximum(
        jnp.dot(cb, a, preferred_element_type=jnp.float32, precision=lax.Precision.HIGHEST) + bm_ref[...], 0.0)
    gid_b = g_ref[0, 0, :]
    gm = (gid_b[None, :]
          == lax.broadcasted_iota(jnp.int32, (G, bn), 0)).astype(jnp.float32)
    p = jnp.dot(gm, y, preferred_element_type=jnp.float32, precision=lax.Precision.HIGHEST)

    @pl.when(i == 0)
    def _():
        pooled[...] = p

    @pl.when(i > 0)
    def _():
        pooled[...] += p

    @pl.when(i == pl.num_programs(0) - 1)
    def _():
        out_ref[...] = jnp.dot(pooled[...], wr_ref[...],
                               preferred_element_type=jnp.float32, precision=lax.Precision.HIGHEST)


def kernel(feats, edge_index, graph_ids, emb_table, W_mlp, b_mlp, W_reg):
    pad = E_PAD - E
    src = jnp.concatenate([edge_index[0], jnp.zeros((pad,), jnp.int32)])
    dst = jnp.concatenate([edge_index[1], jnp.full((pad,), N, jnp.int32)])

    craw = _get_sc_hist()(src, dst, feats, jnp.zeros((CSZ,), jnp.float32))
    c2 = craw.reshape(NC, CROWS, V)

    rpad = CROWS - N
    feats_p = jnp.concatenate(
        [feats, jnp.zeros((rpad,), jnp.int32)]).reshape(2, 1, CROWS // 2)
    gids_p = jnp.concatenate(
        [graph_ids, jnp.full((rpad,), G, jnp.int32)]).reshape(2, 1, CROWS // 2)

    nb = 2
    bn = CROWS // nb
    out = pl.pallas_call(
        _tc_body,
        grid=(nb,),
        in_specs=[
            pl.BlockSpec((NC, bn, V), lambda i: (0, i, 0)),
            pl.BlockSpec((1, 1, bn), lambda i: (i, 0, 0)),
            pl.BlockSpec((1, 1, bn), lambda i: (i, 0, 0)),
            pl.BlockSpec((V, H), lambda i: (0, 0)),
            pl.BlockSpec((H, H), lambda i: (0, 0)),
            pl.BlockSpec((1, H), lambda i: (0, 0)),
            pl.BlockSpec((H, OUT), lambda i: (0, 0)),
        ],
        out_specs=pl.BlockSpec((G, OUT), lambda i: (0, 0)),
        out_shape=jax.ShapeDtypeStruct((G, OUT), jnp.float32),
        scratch_shapes=[pltpu.VMEM((G, H), jnp.float32)],
    )(c2, feats_p, gids_p, emb_table, W_mlp, b_mlp.reshape(1, H), W_reg)
    return out


# speedup vs baseline: 27.3982x; 1.2375x over previous
import functools

import jax
import jax.numpy as jnp
from jax import lax
from jax.experimental import pallas as pl
from jax.experimental.pallas import tpu as pltpu
from jax.experimental.pallas import tpu_sc as plsc

N = 10000
E = 320000
H = 128
V = 128
G = 64
OUT = 1

NC = 2
NS = 16
NW = NC * NS
EPP = E // NW
CHUNK = 80
NCH = EPP // CHUNK
CSZ = N * V
STRIPE = CSZ // NS
ZB = 2048
ZCP = 2000
GRP = 5


def _sc_hist_body(edge_hbm, feats_hbm, out_hbm,
                  src_v, dst_v, feats_v, idx_v, ones_v, zbuf, cs, sem, sem2):
    c = lax.axis_index("c")
    s = lax.axis_index("s")
    wid = c * NS + s

    cp_src = pltpu.async_copy(edge_hbm.at[pl.ds(wid * EPP, EPP)], src_v, sem2)
    cp_dst = pltpu.async_copy(edge_hbm.at[pl.ds(E + wid * EPP, EPP)], dst_v, sem2)
    cp_f = pltpu.async_copy(feats_hbm, feats_v, sem2)

    def z16(i, carry):
        zbuf[pl.ds(i * 16, 16)] = jnp.zeros((16,), jnp.float32)
        return carry
    lax.fori_loop(0, ZB // 16, z16, 0)
    for u in range(CHUNK // 16):
        ones_v[pl.ds(u * 16, 16)] = jnp.full((16,), 1.0, jnp.float32)

    off = s * STRIPE
    for q in range(STRIPE // ZCP):
        pltpu.sync_copy(zbuf.at[pl.ds(0, ZCP)], cs.at[pl.ds(off + q * ZCP, ZCP)])

    cp_src.wait()
    cp_dst.wait()
    cp_f.wait()

    def idx_row(j, carry):
        for k in range(CHUNK // 16):
            base = pl.ds(j * CHUNK + k * 16, 16)
            fv = plsc.load_gather(feats_v, [src_v[base]])
            idx_v[j, pl.ds(k * 16, 16)] = dst_v[base] * V + fv
        return carry
    lax.fori_loop(0, NCH, idx_row, 0)

    plsc.subcore_barrier()

    def scat_grp(g, carry):
        cps = []
        for u in range(GRP):
            j = g * GRP + u
            cps.append(pltpu.async_copy(
                ones_v, cs.at[idx_v.at[j]], sem, add=True))
        for cp in cps:
            cp.wait()
        return carry
    lax.fori_loop(0, NCH // GRP, scat_grp, 0)

    plsc.subcore_barrier()

    pltpu.sync_copy(cs.at[pl.ds(off, STRIPE)], out_hbm.at[c, pl.ds(off, STRIPE)])


@functools.cache
def _get_sc_hist():
    return pl.kernel(
        _sc_hist_body,
        out_type=jax.ShapeDtypeStruct((NC, CSZ), jnp.float32),
        mesh=plsc.VectorSubcoreMesh(core_axis_name="c", subcore_axis_name="s"),
        compiler_params=pltpu.CompilerParams(needs_layout_passes=False),
        scratch_types=[
            pltpu.VMEM((EPP,), jnp.int32),
            pltpu.VMEM((EPP,), jnp.int32),
            pltpu.VMEM((N,), jnp.int32),
            pltpu.VMEM((NCH, CHUNK), jnp.int32),
            pltpu.VMEM((CHUNK,), jnp.float32),
            pltpu.VMEM((ZB,), jnp.float32),
            pltpu.VMEM_SHARED((CSZ,), jnp.float32),
            pltpu.SemaphoreType.DMA,
            pltpu.SemaphoreType.DMA,
        ],
    )


def _tc_body(c_ref, f_ref, g_ref, emb_ref, wm_ref, bm_ref, wr_ref,
             out_ref, pooled):
    i = pl.program_id(0)
    bn = f_ref.shape[2]
    feats_b = f_ref[0, 0, :]
    oh = (feats_b[:, None]
          == lax.broadcasted_iota(jnp.int32, (bn, V), 1)).astype(jnp.float32)
    cb = c_ref[0] + c_ref[1] + oh
    a = jnp.dot(emb_ref[...], wm_ref[...], preferred_element_type=jnp.float32,
                precision=lax.Precision.HIGHEST)
    y = jnp.maximum(
        jnp.dot(cb, a, preferred_element_type=jnp.float32,
                precision=lax.Precision.HIGHEST) + bm_ref[...], 0.0)
    gid_b = g_ref[0, 0, :]
    gm = (gid_b[None, :]
          == lax.broadcasted_iota(jnp.int32, (G, bn), 0)).astype(jnp.float32)
    p = jnp.dot(gm, y, preferred_element_type=jnp.float32,
                precision=lax.Precision.HIGHEST)

    @pl.when(i == 0)
    def _():
        pooled[...] = p

    @pl.when(i > 0)
    def _():
        pooled[...] += p

    @pl.when(i == pl.num_programs(0) - 1)
    def _():
        out_ref[...] = jnp.dot(pooled[...], wr_ref[...],
                               preferred_element_type=jnp.float32,
                               precision=lax.Precision.HIGHEST)


def kernel(feats, edge_index, graph_ids, emb_table, W_mlp, b_mlp, W_reg):
    craw = _get_sc_hist()(edge_index.reshape(2 * E), feats)
    c2 = craw.reshape(NC, N, V)

    feats_p = feats.reshape(2, 1, N // 2)
    gids_p = graph_ids.reshape(2, 1, N // 2)

    nb = 2
    bn = N // nb
    out = pl.pallas_call(
        _tc_body,
        grid=(nb,),
        in_specs=[
            pl.BlockSpec((NC, bn, V), lambda i: (0, i, 0)),
            pl.BlockSpec((1, 1, bn), lambda i: (i, 0, 0)),
            pl.BlockSpec((1, 1, bn), lambda i: (i, 0, 0)),
            pl.BlockSpec((V, H), lambda i: (0, 0)),
            pl.BlockSpec((H, H), lambda i: (0, 0)),
            pl.BlockSpec((1, H), lambda i: (0, 0)),
            pl.BlockSpec((H, OUT), lambda i: (0, 0)),
        ],
        out_specs=pl.BlockSpec((G, OUT), lambda i: (0, 0)),
        out_shape=jax.ShapeDtypeStruct((G, OUT), jnp.float32),
        scratch_shapes=[pltpu.VMEM((G, H), jnp.float32)],
    )(c2, feats_p, gids_p, emb_table, W_mlp, b_mlp.reshape(1, H), W_reg)
    return out


# speedup vs baseline: 31.0671x; 1.1339x over previous
import functools

import jax
import jax.numpy as jnp
from jax import lax
from jax.experimental import pallas as pl
from jax.experimental.pallas import tpu as pltpu
from jax.experimental.pallas import tpu_sc as plsc

N = 10000
E = 320000
H = 128
V = 128
G = 64
OUT = 1

NC = 2
NS = 16
NW = NC * NS
EPP = E // NW
CHUNK = 80
NCH = EPP // CHUNK
CSZ = N * V
STRIPE = CSZ // NS
ZB = 2048
ZCP = 2000
GRP = 5


def _sc_hist_body(edge_hbm, feats_hbm, out_hbm,
                  src_v, dst_v, feats_v, idx_v, ones_v, zbuf, cs, sem, sem2):
    c = lax.axis_index("c")
    s = lax.axis_index("s")
    wid = c * NS + s

    cp_src = pltpu.async_copy(edge_hbm.at[pl.ds(wid * EPP, EPP)], src_v, sem2)
    cp_dst = pltpu.async_copy(edge_hbm.at[pl.ds(E + wid * EPP, EPP)], dst_v, sem2)
    cp_f = pltpu.async_copy(feats_hbm, feats_v, sem2)

    def z16(i, carry):
        zbuf[pl.ds(i * 16, 16)] = jnp.zeros((16,), jnp.float32)
        return carry
    lax.fori_loop(0, ZB // 16, z16, 0)
    for u in range(CHUNK // 16):
        ones_v[pl.ds(u * 16, 16)] = jnp.full((16,), 1.0, jnp.float32)

    off = s * STRIPE
    for q in range(STRIPE // ZCP):
        pltpu.sync_copy(zbuf.at[pl.ds(0, ZCP)], cs.at[pl.ds(off + q * ZCP, ZCP)])

    cp_src.wait()
    cp_dst.wait()
    cp_f.wait()

    def idx_row(j, carry):
        for k in range(CHUNK // 16):
            base = pl.ds(j * CHUNK + k * 16, 16)
            fv = plsc.load_gather(feats_v, [src_v[base]])
            idx_v[j, pl.ds(k * 16, 16)] = dst_v[base] * V + fv
        return carry
    lax.fori_loop(0, NCH, idx_row, 0)

    plsc.subcore_barrier()

    def scat_grp(g, carry):
        cps = []
        for u in range(GRP):
            j = g * GRP + u
            cps.append(pltpu.async_copy(
                ones_v, cs.at[idx_v.at[j]], sem, add=True))
        for cp in cps:
            cp.wait()
        return carry
    lax.fori_loop(0, NCH // GRP, scat_grp, 0)

    plsc.subcore_barrier()

    pltpu.sync_copy(cs.at[pl.ds(off, STRIPE)], out_hbm.at[c, pl.ds(off, STRIPE)])


@functools.cache
def _get_sc_hist():
    return pl.kernel(
        _sc_hist_body,
        out_type=jax.ShapeDtypeStruct((NC, CSZ), jnp.float32),
        mesh=plsc.VectorSubcoreMesh(core_axis_name="c", subcore_axis_name="s"),
        compiler_params=pltpu.CompilerParams(needs_layout_passes=False),
        scratch_types=[
            pltpu.VMEM((EPP,), jnp.int32),
            pltpu.VMEM((EPP,), jnp.int32),
            pltpu.VMEM((N,), jnp.int32),
            pltpu.VMEM((NCH, CHUNK), jnp.int32),
            pltpu.VMEM((CHUNK,), jnp.float32),
            pltpu.VMEM((ZB,), jnp.float32),
            pltpu.VMEM_SHARED((CSZ,), jnp.float32),
            pltpu.SemaphoreType.DMA,
            pltpu.SemaphoreType.DMA,
        ],
    )


def _tc_body(c_ref, f_ref, g_ref, emb_ref, wm_ref, bm_ref, wr_ref,
             out_ref, pooled):
    i = pl.program_id(0)
    bn = f_ref.shape[2]
    feats_b = f_ref[0, 0, :]
    oh = (feats_b[:, None]
          == lax.broadcasted_iota(jnp.int32, (bn, V), 1)).astype(jnp.float32)
    cb = c_ref[0] + c_ref[1] + oh
    a = jnp.dot(emb_ref[...], wm_ref[...], preferred_element_type=jnp.float32,
                precision=lax.Precision.HIGHEST)
    ah = a.astype(jnp.bfloat16).astype(jnp.float32)
    al = a - ah
    y = jnp.maximum(
        jnp.dot(cb, ah, preferred_element_type=jnp.float32)
        + jnp.dot(cb, al, preferred_element_type=jnp.float32)
        + bm_ref[...], 0.0)
    gid_b = g_ref[0, 0, :]
    gm = (gid_b[None, :]
          == lax.broadcasted_iota(jnp.int32, (G, bn), 0)).astype(jnp.float32)
    yh = y.astype(jnp.bfloat16).astype(jnp.float32)
    yl = y - yh
    p = (jnp.dot(gm, yh, preferred_element_type=jnp.float32)
         + jnp.dot(gm, yl, preferred_element_type=jnp.float32))

    @pl.when(i == 0)
    def _():
        pooled[...] = p

    @pl.when(i > 0)
    def _():
        pooled[...] += p

    @pl.when(i == pl.num_programs(0) - 1)
    def _():
        out_ref[...] = jnp.dot(pooled[...], wr_ref[...],
                               preferred_element_type=jnp.float32,
                               precision=lax.Precision.HIGHEST)


def kernel(feats, edge_index, graph_ids, emb_table, W_mlp, b_mlp, W_reg):
    craw = _get_sc_hist()(edge_index.reshape(2 * E), feats)
    c2 = craw.reshape(NC, N, V)

    feats_p = feats.reshape(2, 1, N // 2)
    gids_p = graph_ids.reshape(2, 1, N // 2)

    nb = 2
    bn = N // nb
    out = pl.pallas_call(
        _tc_body,
        grid=(nb,),
        in_specs=[
            pl.BlockSpec((NC, bn, V), lambda i: (0, i, 0)),
            pl.BlockSpec((1, 1, bn), lambda i: (i, 0, 0)),
            pl.BlockSpec((1, 1, bn), lambda i: (i, 0, 0)),
            pl.BlockSpec((V, H), lambda i: (0, 0)),
            pl.BlockSpec((H, H), lambda i: (0, 0)),
            pl.BlockSpec((1, H), lambda i: (0, 0)),
            pl.BlockSpec((H, OUT), lambda i: (0, 0)),
        ],
        out_specs=pl.BlockSpec((G, OUT), lambda i: (0, 0)),
        out_shape=jax.ShapeDtypeStruct((G, OUT), jnp.float32),
        scratch_shapes=[pltpu.VMEM((G, H), jnp.float32)],
    )(c2, feats_p, gids_p, emb_table, W_mlp, b_mlp.reshape(1, H), W_reg)
    return out


# speedup vs baseline: 31.5584x; 1.0158x over previous
import functools

import jax
import jax.numpy as jnp
from jax import lax
from jax.experimental import pallas as pl
from jax.experimental.pallas import tpu as pltpu
from jax.experimental.pallas import tpu_sc as plsc

N = 10000
E = 320000
H = 128
V = 128
G = 64
OUT = 1

NC = 2
NS = 16
NW = NC * NS
EPP = E // NW
CHUNK = 80
NCH = EPP // CHUNK
CSZ = N * V
STRIPE = CSZ // NS
ZB = 2048
ZCP = 2000
GRP = 25


def _sc_hist_body(edge_hbm, feats_hbm, out_hbm,
                  src_v, dst_v, feats_v, idx_v, ones_v, zbuf, cs, sem, sem2):
    c = lax.axis_index("c")
    s = lax.axis_index("s")
    wid = c * NS + s

    cp_src = pltpu.async_copy(edge_hbm.at[pl.ds(wid * EPP, EPP)], src_v, sem2)
    cp_dst = pltpu.async_copy(edge_hbm.at[pl.ds(E + wid * EPP, EPP)], dst_v, sem2)
    cp_f = pltpu.async_copy(feats_hbm, feats_v, sem2)

    def z16(i, carry):
        zbuf[pl.ds(i * 16, 16)] = jnp.zeros((16,), jnp.float32)
        return carry
    lax.fori_loop(0, ZB // 16, z16, 0)
    for u in range(CHUNK // 16):
        ones_v[pl.ds(u * 16, 16)] = jnp.full((16,), 1.0, jnp.float32)

    off = s * STRIPE
    for q in range(STRIPE // ZCP):
        pltpu.sync_copy(zbuf.at[pl.ds(0, ZCP)], cs.at[pl.ds(off + q * ZCP, ZCP)])

    cp_src.wait()
    cp_dst.wait()
    cp_f.wait()

    def idx_row(j, carry):
        for k in range(CHUNK // 16):
            base = pl.ds(j * CHUNK + k * 16, 16)
            fv = plsc.load_gather(feats_v, [src_v[base]])
            idx_v[j, pl.ds(k * 16, 16)] = dst_v[base] * V + fv
        return carry
    lax.fori_loop(0, NCH, idx_row, 0)

    plsc.subcore_barrier()

    def scat_grp(g, carry):
        cps = []
        for u in range(GRP):
            j = g * GRP + u
            cps.append(pltpu.async_copy(
                ones_v, cs.at[idx_v.at[j]], sem, add=True))
        for cp in cps:
            cp.wait()
        return carry
    lax.fori_loop(0, NCH // GRP, scat_grp, 0)

    plsc.subcore_barrier()

    pltpu.sync_copy(cs.at[pl.ds(off, STRIPE)], out_hbm.at[c, pl.ds(off, STRIPE)])


@functools.cache
def _get_sc_hist():
    return pl.kernel(
        _sc_hist_body,
        out_type=jax.ShapeDtypeStruct((NC, CSZ), jnp.float32),
        mesh=plsc.VectorSubcoreMesh(core_axis_name="c", subcore_axis_name="s"),
        compiler_params=pltpu.CompilerParams(needs_layout_passes=False),
        scratch_types=[
            pltpu.VMEM((EPP,), jnp.int32),
            pltpu.VMEM((EPP,), jnp.int32),
            pltpu.VMEM((N,), jnp.int32),
            pltpu.VMEM((NCH, CHUNK), jnp.int32),
            pltpu.VMEM((CHUNK,), jnp.float32),
            pltpu.VMEM((ZB,), jnp.float32),
            pltpu.VMEM_SHARED((CSZ,), jnp.float32),
            pltpu.SemaphoreType.DMA,
            pltpu.SemaphoreType.DMA,
        ],
    )


def _tc_body(c_ref, f_ref, g_ref, emb_ref, wm_ref, bm_ref, wr_ref,
             out_ref, pooled):
    i = pl.program_id(0)
    bn = f_ref.shape[2]
    feats_b = f_ref[0, 0, :]
    oh = (feats_b[:, None]
          == lax.broadcasted_iota(jnp.int32, (bn, V), 1)).astype(jnp.float32)
    cb = c_ref[0] + c_ref[1] + oh
    a = jnp.dot(emb_ref[...], wm_ref[...], preferred_element_type=jnp.float32,
                precision=lax.Precision.HIGHEST)
    ah = a.astype(jnp.bfloat16).astype(jnp.float32)
    al = a - ah
    y = jnp.maximum(
        jnp.dot(cb, ah, preferred_element_type=jnp.float32)
        + jnp.dot(cb, al, preferred_element_type=jnp.float32)
        + bm_ref[...], 0.0)
    gid_b = g_ref[0, 0, :]
    gm = (gid_b[None, :]
          == lax.broadcasted_iota(jnp.int32, (G, bn), 0)).astype(jnp.float32)
    yh = y.astype(jnp.bfloat16).astype(jnp.float32)
    yl = y - yh
    p = (jnp.dot(gm, yh, preferred_element_type=jnp.float32)
         + jnp.dot(gm, yl, preferred_element_type=jnp.float32))

    @pl.when(i == 0)
    def _():
        pooled[...] = p

    @pl.when(i > 0)
    def _():
        pooled[...] += p

    @pl.when(i == pl.num_programs(0) - 1)
    def _():
        out_ref[...] = jnp.dot(pooled[...], wr_ref[...],
                               preferred_element_type=jnp.float32,
                               precision=lax.Precision.HIGHEST)


def kernel(feats, edge_index, graph_ids, emb_table, W_mlp, b_mlp, W_reg):
    craw = _get_sc_hist()(edge_index.reshape(2 * E), feats)
    c2 = craw.reshape(NC, N, V)

    feats_p = feats.reshape(2, 1, N // 2)
    gids_p = graph_ids.reshape(2, 1, N // 2)

    nb = 2
    bn = N // nb
    out = pl.pallas_call(
        _tc_body,
        grid=(nb,),
        in_specs=[
            pl.BlockSpec((NC, bn, V), lambda i: (0, i, 0)),
            pl.BlockSpec((1, 1, bn), lambda i: (i, 0, 0)),
            pl.BlockSpec((1, 1, bn), lambda i: (i, 0, 0)),
            pl.BlockSpec((V, H), lambda i: (0, 0)),
            pl.BlockSpec((H, H), lambda i: (0, 0)),
            pl.BlockSpec((1, H), lambda i: (0, 0)),
            pl.BlockSpec((H, OUT), lambda i: (0, 0)),
        ],
        out_specs=pl.BlockSpec((G, OUT), lambda i: (0, 0)),
        out_shape=jax.ShapeDtypeStruct((G, OUT), jnp.float32),
        scratch_shapes=[pltpu.VMEM((G, H), jnp.float32)],
    )(c2, feats_p, gids_p, emb_table, W_mlp, b_mlp.reshape(1, H), W_reg)
    return out
